# baseline (speedup 1.0000x reference)
import functools

import jax
import jax.numpy as jnp
from jax import lax
from jax.experimental import pallas as pl
from jax.experimental.pallas import tpu as pltpu
from jax.experimental.pallas import tpu_sc as plsc

N_ROWS = 320000
D = 128
N_SEG = 512
N_WORKERS = 32
ROWS_PER_W = N_ROWS // N_WORKERS
CHUNK = 80
CHUNKS_PER_W = ROWS_PER_W // CHUNK
SEG_PER_SUB = N_SEG // 16
NBUF = 4
MERGE_B = N_SEG // 128


def _sc_body(h_hbm, idx_hbm, iota_hbm, out_hbm, *sc):
    bufs = sc[:NBUF]
    idx_v, idx_id, zero_v, mbuf, slab, acc_sh = sc[NBUF:NBUF + 6]
    lsems = sc[NBUF + 6:2 * NBUF + 6]
    ssems = sc[2 * NBUF + 6:]
    core = lax.axis_index("c")
    sub = lax.axis_index("s")
    wid = core * 16 + sub
    row_base = wid * ROWS_PER_W

    def zrow(r, _):
        for k in range(D // 16):
            zero_v[r, pl.ds(k * 16, 16)] = jnp.zeros((16,), jnp.float32)
        return 0
    lax.fori_loop(0, SEG_PER_SUB, zrow, 0)
    slab_base = (sub // 2) * N_SEG
    half = sub % 2
    for k in range(N_SEG // SEG_PER_SUB // 2):
        pltpu.sync_copy(
            zero_v,
            slab.at[pl.ds(slab_base + half * (N_SEG // 2)
                          + k * SEG_PER_SUB, SEG_PER_SUB)])
    pltpu.sync_copy(zero_v, acc_sh.at[pl.ds(sub * SEG_PER_SUB, SEG_PER_SUB)])

    pltpu.sync_copy(idx_hbm.at[wid], idx_v)
    pltpu.sync_copy(iota_hbm, idx_id)

    def rebase(r, _):
        for g in range(CHUNK // 16):
            idx_v[r, pl.ds(g * 16, 16)] = (
                idx_v[r, pl.ds(g * 16, 16)] + slab_base)
        return 0
    lax.fori_loop(0, CHUNKS_PER_W, rebase, 0)

    plsc.subcore_barrier()

    def load_start(c, b):
        pltpu.async_copy(
            h_hbm.at[pl.ds(row_base + c * CHUNK, CHUNK)], bufs[b], lsems[b])

    def load_wait(c, b):
        pltpu.make_async_copy(
            h_hbm.at[pl.ds(row_base + c * CHUNK, CHUNK)], bufs[b],
            lsems[b]).wait()

    def scat_start(c, b):
        pltpu.async_copy(bufs[b], slab.at[idx_v.at[c]], ssems[b], add=True)

    def scat_wait(c, b):
        pltpu.make_async_copy(bufs[b], slab.at[idx_v.at[c]], ssems[b]).wait()

    MAIN = (CHUNKS_PER_W // NBUF) * NBUF

    load_start(0, 0)
    load_start(1, 1)

    def ring_body(i, _):
        for b in range(NBUF):
            c = i * NBUF + b

            def prefetch(c=c, bn=(b + 2) % NBUF):
                @pl.when(c >= 2)
                def _():
                    scat_wait(c - 2, bn)
                load_start(c + 2, bn)

            if b == 0:
                prefetch()
            else:
                pl.when(c + 2 < CHUNKS_PER_W)(prefetch)
            load_wait(c, b)
            scat_start(c, b)
        return 0

    lax.fori_loop(0, MAIN // NBUF, ring_body, 0)

    for c in range(MAIN, CHUNKS_PER_W):
        load_wait(c, c % NBUF)
        scat_start(c, c % NBUF)
    for c in range(CHUNKS_PER_W - NBUF, CHUNKS_PER_W):
        scat_wait(c, c % NBUF)

    plsc.subcore_barrier()

    for k in range(MERGE_B // 2):
        k_eff = half * (MERGE_B // 2) + k
        pltpu.sync_copy(
            slab.at[pl.ds(slab_base + half * (N_SEG // 2) + k * 128, 128)],
            mbuf)
        pltpu.sync_copy(mbuf, acc_sh.at[idx_id.at[k_eff]], add=True)

    plsc.subcore_barrier()

    pltpu.sync_copy(
        acc_sh.at[pl.ds(sub * SEG_PER_SUB, SEG_PER_SUB)],
        out_hbm.at[core, pl.ds(sub * SEG_PER_SUB, SEG_PER_SUB)])


_sc_segsum = functools.partial(
    pl.kernel,
    out_type=jax.ShapeDtypeStruct((2, N_SEG, D), jnp.float32),
    mesh=plsc.VectorSubcoreMesh(core_axis_name="c", subcore_axis_name="s"),
    scratch_types=(
        [pltpu.VMEM((CHUNK, D), jnp.float32) for _ in range(NBUF)]
        + [
            pltpu.VMEM((CHUNKS_PER_W, CHUNK), jnp.int32),
            pltpu.VMEM((MERGE_B, 128), jnp.int32),
            pltpu.VMEM((SEG_PER_SUB, D), jnp.float32),
            pltpu.VMEM((128, D), jnp.float32),
            pltpu.VMEM_SHARED((8 * N_SEG, D), jnp.float32),
            pltpu.VMEM_SHARED((N_SEG, D), jnp.float32),
        ]
        + [pltpu.SemaphoreType.DMA for _ in range(2 * NBUF)]
    ),
)(_sc_body)


def _merge_body(p_ref, o_ref):
    o_ref[...] = p_ref[0] + p_ref[1]


def _merge(partials):
    return pl.pallas_call(
        _merge_body,
        out_shape=jax.ShapeDtypeStruct((N_SEG, D), jnp.float32),
    )(partials)


@jax.jit
def kernel(h, index):
    idx = index.astype(jnp.int32).reshape(N_WORKERS, CHUNKS_PER_W, CHUNK)
    iota = jnp.arange(N_SEG, dtype=jnp.int32).reshape(MERGE_B, 128)
    partials = _sc_segsum(h, idx, iota)
    return _merge(partials)
